# baseline (speedup 1.0000x reference)
import functools

import jax
import jax.numpy as jnp
from jax.experimental import pallas as pl
from jax.experimental.pallas import tpu as pltpu

NS = [10000, 2500, 640, 160, 40]
ES = [160000, 40000, 10000, 2500, 600]
G = 8
K = 5
B = K ** 3


def _spline_conv(x, edge_index, pseudo, W, R, b):
    N, Cin = x.shape
    src, dst = edge_index[0], edge_index[1]
    x_src = x[src]
    p = pseudo * (K - 1)
    lo = jnp.clip(jnp.floor(p), 0, K - 2).astype(jnp.int32)
    fr = p - lo
    acc = jnp.zeros((N * B, Cin), dtype=x.dtype)
    for bits in range(8):
        b0, b1, b2 = (bits >> 2) & 1, (bits >> 1) & 1, bits & 1
        idx = (lo[:, 0] + b0) * (K * K) + (lo[:, 1] + b1) * K + (lo[:, 2] + b2)
        c0 = fr[:, 0] if b0 else 1.0 - fr[:, 0]
        c1 = fr[:, 1] if b1 else 1.0 - fr[:, 1]
        c2 = fr[:, 2] if b2 else 1.0 - fr[:, 2]
        coef = c0 * c1 * c2
        seg = dst * B + idx
        acc = acc + jax.ops.segment_sum(coef[:, None] * x_src, seg, num_segments=N * B)
    A = acc.reshape(N, B, Cin)
    out = jnp.einsum('nbi,bio->no', A, W)
    deg = jax.ops.segment_sum(jnp.ones(dst.shape, dtype=x.dtype), dst, num_segments=N)
    out = out / jnp.clip(deg, 1.0)[:, None]
    return out + x @ R + b


def _graph_mean(f, batch, num_graphs):
    s = jax.ops.segment_sum(f, batch, num_segments=num_graphs)
    cnt = jax.ops.segment_sum(jnp.ones(batch.shape, dtype=f.dtype), batch, num_segments=num_graphs)
    return s / jnp.clip(cnt, 1.0)[:, None]


def _voxel_max_pool(inp, weight, cluster, n_next):
    N = inp.shape[0]
    m = jax.ops.segment_max(weight, cluster, num_segments=n_next)
    is_max = weight >= m[cluster]
    cand = jnp.where(is_max, jnp.arange(N), N)
    sel = jax.ops.segment_min(cand, cluster, num_segments=n_next)
    sel = jnp.clip(sel, 0, N - 1)
    return inp[sel]


def _fc_kernel(xcat_ref, w_ref, b_ref, out_ref):
    logits = jnp.dot(xcat_ref[...], w_ref[...], preferred_element_type=jnp.float32)
    logits = logits + b_ref[...][None, :]
    mx = jnp.max(logits, axis=1, keepdims=True)
    sh = logits - mx
    lse = jnp.log(jnp.sum(jnp.exp(sh), axis=1, keepdims=True))
    out_ref[...] = sh - lse


def kernel(x, edge_index1, pseudo1, batch1, cluster1, edge_index2, pseudo2, batch2, cluster2, edge_index3, pseudo3, batch3, cluster3, edge_index4, pseudo4, batch4, cluster4, edge_index5, pseudo5, batch5, W1, R1, b1, W12, R12, b12, W2, R2, b2, W22, R22, b22, W3, R3, b3, W32, R32, b32, W4, R4, b4, W42, R42, b42, W5, R5, b5, W52, R52, b52, fcW, fcb):
    d = dict(
        edge_index1=edge_index1, pseudo1=pseudo1, batch1=batch1, cluster1=cluster1,
        edge_index2=edge_index2, pseudo2=pseudo2, batch2=batch2, cluster2=cluster2,
        edge_index3=edge_index3, pseudo3=pseudo3, batch3=batch3, cluster3=cluster3,
        edge_index4=edge_index4, pseudo4=pseudo4, batch4=batch4, cluster4=cluster4,
        edge_index5=edge_index5, pseudo5=pseudo5, batch5=batch5,
    )
    convs = {
        "1": (W1, R1, b1), "12": (W12, R12, b12),
        "2": (W2, R2, b2), "22": (W22, R22, b22),
        "3": (W3, R3, b3), "32": (W32, R32, b32),
        "4": (W4, R4, b4), "42": (W42, R42, b42),
        "5": (W5, R5, b5), "52": (W52, R52, b52),
    }
    res = []
    cur = x
    pairs = [("1", "12"), ("2", "22"), ("3", "32"), ("4", "42"), ("5", "52")]
    for l in range(5):
        a, c = pairs[l]
        ei = d["edge_index%d" % (l + 1)]
        ps = d["pseudo%d" % (l + 1)]
        f = jax.nn.relu(_spline_conv(cur, ei, ps, *convs[a]))
        f = jax.nn.relu(_spline_conv(f, ei, ps, *convs[c]))
        res.append(_graph_mean(f, d["batch%d" % (l + 1)], G))
        if l < 4:
            cur = _voxel_max_pool(cur, f[:, 0], d["cluster%d" % (l + 1)], NS[l + 1])
    xcat = jnp.concatenate(res, axis=1)
    out = pl.pallas_call(
        _fc_kernel,
        out_shape=jax.ShapeDtypeStruct((G, 10), jnp.float32),
    )(xcat, fcW, fcb)
    return out


# speedup vs baseline: 2.9389x; 2.9389x over previous
import functools

import jax
import jax.numpy as jnp
from jax import lax
from jax.experimental import pallas as pl
from jax.experimental.pallas import tpu as pltpu
from jax.experimental.pallas import tpu_sc as plsc

NS = [10000, 2500, 640, 160, 40]
ES = [160000, 40000, 10000, 2500, 600]
G = 8
K = 5
NCORE = 2
NSUB = 16
NW = NCORE * NSUB
LANES = 16


def _ceil_to(a, m):
    return -(-a // m) * m


def _corners(p0, p1, p2):
    q0, q1, q2 = p0 * 4.0, p1 * 4.0, p2 * 4.0
    lo0 = jnp.minimum(jnp.maximum(q0.astype(jnp.int32), 0), 3)
    lo1 = jnp.minimum(jnp.maximum(q1.astype(jnp.int32), 0), 3)
    lo2 = jnp.minimum(jnp.maximum(q2.astype(jnp.int32), 0), 3)
    fr0 = q0 - lo0.astype(jnp.float32)
    fr1 = q1 - lo1.astype(jnp.float32)
    fr2 = q2 - lo2.astype(jnp.float32)
    out = []
    for bits in range(8):
        b0, b1, b2 = (bits >> 2) & 1, (bits >> 1) & 1, bits & 1
        bin_ = (lo0 + b0) * 25 + (lo1 + b1) * 5 + (lo2 + b2)
        c0 = fr0 if b0 else 1.0 - fr0
        c1 = fr1 if b1 else 1.0 - fr1
        c2 = fr2 if b2 else 1.0 - fr2
        out.append((bin_, c0 * c1 * c2))
    return out


def _narrow_sc(Np, Ep, C):
    Wn = C + 16
    Epw = Ep // NW
    groups = Epw // LANES
    r = Np // NSUB
    mesh = plsc.VectorSubcoreMesh(core_axis_name="c", subcore_axis_name="s")

    def body(src_hbm, dst_hbm, p0_hbm, p1_hbm, p2_hbm, x_hbm, t_hbm, out_hbm,
             srcb, dstb, p0b, p1b, p2b, xb, tb, binb, cfb, msgb, zb,
             accum):
        cid = lax.axis_index("c")
        sid = lax.axis_index("s")
        wid = cid * NSUB + sid
        row0 = sid * r
        pltpu.sync_copy(src_hbm.at[pl.ds(wid * Epw, Epw)], srcb)
        pltpu.sync_copy(dst_hbm.at[pl.ds(wid * Epw, Epw)], dstb)
        pltpu.sync_copy(p0_hbm.at[pl.ds(wid * Epw, Epw)], p0b)
        pltpu.sync_copy(p1_hbm.at[pl.ds(wid * Epw, Epw)], p1b)
        pltpu.sync_copy(p2_hbm.at[pl.ds(wid * Epw, Epw)], p2b)
        pltpu.sync_copy(x_hbm, xb)
        pltpu.sync_copy(t_hbm, tb)

        zeros16 = jnp.zeros((16,), jnp.float32)
        iota = lax.iota(jnp.int32, 16)
        onehot = jnp.where(iota == 0, 1.0, 0.0).astype(jnp.float32)
        for i in range(LANES):
            for j in range(C // 16):
                msgb[i, pl.ds(j * 16, 16)] = zeros16
            msgb[i, pl.ds(C, 16)] = onehot
            for j in range(Wn // 16):
                zb[i, pl.ds(j * 16, 16)] = zeros16

        def zinit(i, carry):
            pltpu.sync_copy(zb, accum.at[pl.ds(row0 + i * 16, 16)])
            return carry
        lax.fori_loop(0, r // 16, zinit, 0)
        plsc.subcore_barrier()

        def grp(g, carry):
            base = g * LANES
            sv = srcb[pl.ds(base, 16)]
            dv = dstb[pl.ds(base, 16)]
            pa = p0b[pl.ds(base, 16)]
            pb_ = p1b[pl.ds(base, 16)]
            pc = p2b[pl.ds(base, 16)]
            xs = plsc.load_gather(xb, [sv])
            corners = _corners(pa, pb_, pc)
            for c8, (bin_, coef) in enumerate(corners):
                binb[pl.ds(c8 * 16, 16)] = bin_ * C
                cfb[pl.ds(c8 * 16, 16)] = coef * xs
            for e in range(LANES):
                accs = [None] * (C // 16)
                for c8 in range(8):
                    pos = jnp.full((16,), c8 * 16 + e, jnp.int32)
                    bs = plsc.load_gather(binb, [pos])
                    cs = plsc.load_gather(cfb, [pos])
                    for blk in range(C // 16):
                        v = plsc.load_gather(tb, [bs + (blk * 16) + iota])
                        accs[blk] = (cs * v if accs[blk] is None
                                     else accs[blk] + cs * v)
                for blk in range(C // 16):
                    msgb[e, pl.ds(blk * 16, 16)] = accs[blk]
            pltpu.sync_copy(msgb, accum.at[dv], add=True)
            return carry
        lax.fori_loop(0, groups, grp, 0)

        plsc.subcore_barrier()

        def cpout(i, carry):
            pltpu.sync_copy(accum.at[pl.ds(row0 + i * 16, 16)], zb)
            pltpu.sync_copy(zb, out_hbm.at[wid, pl.ds(i * 16, 16)])
            return carry
        lax.fori_loop(0, r // 16, cpout, 0)

    return pl.kernel(
        body, mesh=mesh,
        compiler_params=pltpu.CompilerParams(
            needs_layout_passes=False, use_tc_tiling_on_sc=False),
        out_type=jax.ShapeDtypeStruct((NW, r, Wn), jnp.float32),
        scratch_types=[
            pltpu.VMEM((Epw,), jnp.int32),
            pltpu.VMEM((Epw,), jnp.int32),
            pltpu.VMEM((Epw,), jnp.float32),
            pltpu.VMEM((Epw,), jnp.float32),
            pltpu.VMEM((Epw,), jnp.float32),
            pltpu.VMEM((Np,), jnp.float32),
            pltpu.VMEM((125 * C,), jnp.float32),
            pltpu.VMEM((8 * LANES,), jnp.int32),
            pltpu.VMEM((8 * LANES,), jnp.float32),
            pltpu.VMEM((LANES, Wn), jnp.float32),
            pltpu.VMEM((16, Wn), jnp.float32),
            pltpu.VMEM_SHARED((Np, Wn), jnp.float32),
        ],
    )


def _wide_sc(Np, Ep, C):
    Epw = Ep // NW
    groups = Epw // LANES
    r = Np // NSUB
    mesh = plsc.VectorSubcoreMesh(core_axis_name="c", subcore_axis_name="s")

    def body(src_hbm, dst_hbm, p0_hbm, p1_hbm, p2_hbm, y_hbm, out_hbm,
             srcb, dstb, p0b, p1b, p2b, gi, cf, rows, msgb, zb, accum):
        cid = lax.axis_index("c")
        sid = lax.axis_index("s")
        wid = cid * NSUB + sid
        row0 = sid * r
        pltpu.sync_copy(src_hbm.at[pl.ds(wid * Epw, Epw)], srcb)
        pltpu.sync_copy(dst_hbm.at[pl.ds(wid * Epw, Epw)], dstb)
        pltpu.sync_copy(p0_hbm.at[pl.ds(wid * Epw, Epw)], p0b)
        pltpu.sync_copy(p1_hbm.at[pl.ds(wid * Epw, Epw)], p1b)
        pltpu.sync_copy(p2_hbm.at[pl.ds(wid * Epw, Epw)], p2b)

        zeros16 = jnp.zeros((16,), jnp.float32)
        iota = lax.iota(jnp.int32, 16)
        for i in range(16):
            for j in range(C // 16):
                zb[i, pl.ds(j * 16, 16)] = zeros16

        def zinit(i, carry):
            pltpu.sync_copy(zb, accum.at[pl.ds(row0 + i * 16, 16)])
            return carry
        lax.fori_loop(0, r // 16, zinit, 0)
        plsc.subcore_barrier()

        def grp(g, carry):
            base = g * LANES
            sv = srcb[pl.ds(base, 16)]
            dv = dstb[pl.ds(base, 16)]
            pa = p0b[pl.ds(base, 16)]
            pb_ = p1b[pl.ds(base, 16)]
            pc = p2b[pl.ds(base, 16)]
            corners = _corners(pa, pb_, pc)
            for c8, (bin_, coef) in enumerate(corners):
                gi[pl.ds(c8 * 16, 16)] = bin_ * Np + sv
                cf[pl.ds(c8 * 16, 16)] = coef
            pltpu.sync_copy(y_hbm.at[gi], rows)
            for e in range(LANES):
                accs = [None] * (C // 16)
                for c8 in range(8):
                    cs = plsc.load_gather(
                        cf, [jnp.full((16,), c8 * 16 + e, jnp.int32)])
                    for blk in range(C // 16):
                        v = rows[c8 * 16 + e, pl.ds(blk * 16, 16)]
                        accs[blk] = (cs * v if accs[blk] is None
                                     else accs[blk] + cs * v)
                for blk in range(C // 16):
                    msgb[e, pl.ds(blk * 16, 16)] = accs[blk]
            pltpu.sync_copy(msgb, accum.at[dv], add=True)
            return carry
        lax.fori_loop(0, groups, grp, 0)

        plsc.subcore_barrier()

        def cpout(i, carry):
            pltpu.sync_copy(accum.at[pl.ds(row0 + i * 16, 16)], zb)
            pltpu.sync_copy(zb, out_hbm.at[wid, pl.ds(i * 16, 16)])
            return carry
        lax.fori_loop(0, r // 16, cpout, 0)

    return pl.kernel(
        body, mesh=mesh,
        compiler_params=pltpu.CompilerParams(
            needs_layout_passes=False, use_tc_tiling_on_sc=False),
        out_type=jax.ShapeDtypeStruct((NW, r, C), jnp.float32),
        scratch_types=[
            pltpu.VMEM((Epw,), jnp.int32),
            pltpu.VMEM((Epw,), jnp.int32),
            pltpu.VMEM((Epw,), jnp.float32),
            pltpu.VMEM((Epw,), jnp.float32),
            pltpu.VMEM((Epw,), jnp.float32),
            pltpu.VMEM((8 * LANES,), jnp.int32),
            pltpu.VMEM((8 * LANES,), jnp.float32),
            pltpu.VMEM((8 * LANES, C), jnp.float32),
            pltpu.VMEM((LANES, C), jnp.float32),
            pltpu.VMEM((16, C), jnp.float32),
            pltpu.VMEM_SHARED((Np, C), jnp.float32),
        ],
    )


def _ymat_tc(Np, Cin, C):
    BM = 256

    def body(x_ref, w_ref, o_ref):
        o_ref[...] = jnp.dot(
            x_ref[...], w_ref[0], preferred_element_type=jnp.float32)[None]

    return pl.pallas_call(
        body,
        grid=(125, Np // BM),
        in_specs=[
            pl.BlockSpec((BM, Cin), lambda b, m: (m, 0)),
            pl.BlockSpec((1, Cin, C), lambda b, m: (b, 0, 0)),
        ],
        out_specs=pl.BlockSpec((1, BM, C), lambda b, m: (b, m, 0)),
        out_shape=jax.ShapeDtypeStruct((125, Np, C), jnp.float32),
    )


def _finish_tc(Np, Cin, C, Wn, narrow):
    def body_narrow(a0_ref, a1_ref, x_ref, r_ref, b_ref, f_ref, deg_ref):
        deg = a0_ref[:, C] + a1_ref[:, C]
        d = jnp.maximum(deg, 1.0)
        acc = a0_ref[:, :C] + a1_ref[:, :C]
        f = acc / d[:, None] + jnp.dot(
            x_ref[...], r_ref[...], preferred_element_type=jnp.float32)
        f = f + b_ref[...]
        f_ref[...] = jnp.maximum(f, 0.0)
        deg_ref[...] = deg[:, None]

    def body_wide(a0_ref, a1_ref, x_ref, r_ref, b_ref, deg_ref, f_ref):
        d = jnp.maximum(deg_ref[:, 0], 1.0)
        acc = a0_ref[:, :C] + a1_ref[:, :C]
        f = acc / d[:, None] + jnp.dot(
            x_ref[...], r_ref[...], preferred_element_type=jnp.float32)
        f = f + b_ref[...]
        f_ref[...] = jnp.maximum(f, 0.0)

    if narrow:
        return pl.pallas_call(
            body_narrow,
            out_shape=(jax.ShapeDtypeStruct((Np, C), jnp.float32),
                       jax.ShapeDtypeStruct((Np, 1), jnp.float32)),
        )
    return pl.pallas_call(
        body_wide,
        out_shape=jax.ShapeDtypeStruct((Np, C), jnp.float32),
    )


def _mean_tc(Np, C):
    def body(f_ref, b_ref, o_ref):
        f = f_ref[...]
        bi = b_ref[...]
        rows = []
        for g in range(G):
            m = (bi == g).astype(jnp.float32)
            s = jnp.sum(m * f, axis=0)
            c = jnp.maximum(jnp.sum(m), 1.0)
            rows.append(s / c)
        o_ref[...] = jnp.stack(rows, axis=0)

    return pl.pallas_call(
        body,
        out_shape=jax.ShapeDtypeStruct((G, C), jnp.float32),
    )


def _fc_kernel(xcat_ref, w_ref, b_ref, out_ref):
    logits = jnp.dot(xcat_ref[...], w_ref[...],
                     preferred_element_type=jnp.float32)
    logits = logits + b_ref[...][None, :]
    mx = jnp.max(logits, axis=1, keepdims=True)
    sh = logits - mx
    lse = jnp.log(jnp.sum(jnp.exp(sh), axis=1, keepdims=True))
    out_ref[...] = sh - lse


def _pool_jnp(cur, w, cluster, n_next):
    n = cur.shape[0]
    m = jax.ops.segment_max(w, cluster, num_segments=n_next)
    is_max = w >= m[cluster]
    cand = jnp.where(is_max, jnp.arange(n), n)
    sel = jnp.clip(jax.ops.segment_min(cand, cluster, num_segments=n_next),
                   0, n - 1)
    return cur[sel]


def kernel(x, edge_index1, pseudo1, batch1, cluster1,
           edge_index2, pseudo2, batch2, cluster2,
           edge_index3, pseudo3, batch3, cluster3,
           edge_index4, pseudo4, batch4, cluster4,
           edge_index5, pseudo5, batch5,
           W1, R1, b1, W12, R12, b12,
           W2, R2, b2, W22, R22, b22,
           W3, R3, b3, W32, R32, b32,
           W4, R4, b4, W42, R42, b42,
           W5, R5, b5, W52, R52, b52,
           fcW, fcb):
    d = dict(locals())
    convs = {
        "1": (W1, R1, b1), "12": (W12, R12, b12),
        "2": (W2, R2, b2), "22": (W22, R22, b22),
        "3": (W3, R3, b3), "32": (W32, R32, b32),
        "4": (W4, R4, b4), "42": (W42, R42, b42),
        "5": (W5, R5, b5), "52": (W52, R52, b52),
    }
    pairs = [("1", "12"), ("2", "22"), ("3", "32"), ("4", "42"), ("5", "52")]
    cur = x[:, 0]
    res = []
    for l in range(5):
        N, E = NS[l], ES[l]
        C1 = 32 if l == 0 else 64
        Np = _ceil_to(N + 1, 256)
        Ep = _ceil_to(E, NW * LANES)
        a, c = pairs[l]
        Wa, Ra, ba = convs[a]
        Wc, Rc, bc = convs[c]
        ei = d["edge_index%d" % (l + 1)]
        ps = d["pseudo%d" % (l + 1)]
        src = jnp.pad(ei[0].astype(jnp.int32), (0, Ep - E))
        dst = jnp.pad(ei[1].astype(jnp.int32), (0, Ep - E),
                      constant_values=Np - 1)
        p0 = jnp.pad(ps[:, 0], (0, Ep - E))
        p1 = jnp.pad(ps[:, 1], (0, Ep - E))
        p2 = jnp.pad(ps[:, 2], (0, Ep - E))
        xp = jnp.pad(cur, (0, Np - N))

        t_flat = Wa[:, 0, :].reshape(-1)
        o1 = _narrow_sc(Np, Ep, C1)(src, dst, p0, p1, p2, xp, t_flat)
        o1 = o1.reshape(NCORE, Np, C1 + 16)
        f1, deg = _finish_tc(Np, 1, C1, C1 + 16, True)(
            o1[0], o1[1], xp[:, None], Ra, ba[None, :])

        y = _ymat_tc(Np, C1, 64)(f1, Wc)
        yf = y.reshape(125 * Np, 64)
        o2 = _wide_sc(Np, Ep, 64)(src, dst, p0, p1, p2, yf)
        o2 = o2.reshape(NCORE, Np, 64)
        f2 = _finish_tc(Np, C1, 64, 64, False)(
            o2[0], o2[1], f1, Rc, bc[None, :], deg)

        bp = jnp.pad(d["batch%d" % (l + 1)].astype(jnp.int32), (0, Np - N),
                     constant_values=G)[:, None]
        res.append(_mean_tc(Np, 64)(f2, bp))
        if l < 4:
            cur = _pool_jnp(cur, f2[:N, 0],
                            d["cluster%d" % (l + 1)].astype(jnp.int32),
                            NS[l + 1])
    xcat = jnp.concatenate(res, axis=1)
    out = pl.pallas_call(
        _fc_kernel,
        out_shape=jax.ShapeDtypeStruct((G, 10), jnp.float32),
    )(xcat, fcW, fcb)
    return out
